# baseline (speedup 1.0000x reference)
import functools

import jax
import jax.numpy as jnp
from jax import lax
from jax.experimental import pallas as pl
from jax.experimental.pallas import tpu as pltpu
from jax.experimental.pallas import tpu_sc as plsc

N = 10000
NP = 10240
FEATS = 128
E = 320000
NC, NS, LANES = 2, 16, 16
NPT = NP // NS
SUB = 128
NSUB = NPT // SUB
ECHUNK = 128
EPT = E // NS
EPTP = 20480
NCHUNK = EPTP // ECHUNK
PADV = 10016
EPW = E // (NC * NS)
DEPTH = 10

_mesh = functools.partial(
    plsc.VectorSubcoreMesh, core_axis_name="c", subcore_axis_name="s",
    num_cores=NC, num_subcores=NS)

f32 = jnp.float32
i32 = jnp.int32


def _deg_body(dst_h, out_h, idx_v, hist_v, cbuf, dbuf, hall_s):
    c = lax.axis_index("c")
    s = lax.axis_index("s")
    w = s * NC + c
    zero16 = jnp.zeros((LANES,), f32)
    one16 = jnp.ones((LANES,), f32)

    @pl.loop(0, NP // LANES)
    def _zero(r):
        hist_v[pl.ds(r * LANES, LANES)] = zero16

    pltpu.sync_copy(dst_h.at[pl.ds(w * EPW, EPW)], idx_v)

    @pl.loop(0, EPW // LANES)
    def _scat(i):
        idx16 = idx_v[pl.ds(i * LANES, LANES)]
        plsc.addupdate_scatter(hist_v, [idx16], one16)

    pltpu.sync_copy(hist_v, hall_s.at[s])
    plsc.subcore_barrier()

    @pl.loop(0, NPT // LANES)
    def _zd(r):
        dbuf[pl.ds(r * LANES, LANES)] = zero16

    for t in range(NS):
        pltpu.sync_copy(hall_s.at[t].at[pl.ds(s * NPT, NPT)], cbuf)

        @pl.loop(0, NPT // LANES)
        def _acc(r):
            sl = pl.ds(r * LANES, LANES)
            dbuf[sl] = dbuf[sl] + cbuf[sl]

    pltpu.sync_copy(dbuf, out_h.at[c].at[s])


def _deg_call(dst):
    k = pl.kernel(
        _deg_body,
        out_type=jax.ShapeDtypeStruct((NC, NS, NPT), f32),
        mesh=_mesh(),
        compiler_params=pltpu.CompilerParams(needs_layout_passes=False),
        scratch_types=[
            pltpu.VMEM((EPW,), i32),
            pltpu.VMEM((NP,), f32),
            pltpu.VMEM((NPT,), f32),
            pltpu.VMEM((NPT,), f32),
            pltpu.VMEM_SHARED((NS, NP), f32),
        ],
    )
    return k(dst)


def _prep_body(d0_ref, d1_ref, x_ref, dis_ref, w_ref, g0_ref):
    deg = d0_ref[:] + d1_ref[:]
    dis = jnp.where(deg > 0, lax.rsqrt(jnp.maximum(deg, 1e-12)), 0.0)
    dis_ref[:] = dis
    w_ref[:] = dis * dis
    g0_ref[:] = x_ref[:] * dis


def _prep_call(d0, d1, xp):
    bn = 256
    grid = NP // bn
    return pl.pallas_call(
        _prep_body,
        grid=(grid,),
        in_specs=[
            pl.BlockSpec((bn, 1), lambda i: (i, 0)),
            pl.BlockSpec((bn, 1), lambda i: (i, 0)),
            pl.BlockSpec((bn, FEATS), lambda i: (i, 0)),
        ],
        out_specs=[
            pl.BlockSpec((bn, 1), lambda i: (i, 0)),
            pl.BlockSpec((bn, 1), lambda i: (i, 0)),
            pl.BlockSpec((bn, FEATS), lambda i: (i, 0)),
        ],
        out_shape=[
            jax.ShapeDtypeStruct((NP, 1), f32),
            jax.ShapeDtypeStruct((NP, 1), f32),
            jax.ShapeDtypeStruct((NP, FEATS), f32),
        ],
    )(d0, d1, xp)


def _diff_body(W, RING, GAHEAD, g0_h, u_h, w_h, dis_h, src_h, dst_h, out_h, g_h,
               acc_s, src_v, dst_v, bufs, zbuf, w_v, dis_v,
               gsems, ssems):
    c = lax.axis_index("c")
    s = lax.axis_index("s")
    assert W % LANES == 0
    FC = W // LANES
    zero16 = jnp.zeros((LANES,), f32)
    buf0, buf1 = bufs[0], bufs[1]

    pltpu.sync_copy(src_h.at[s], src_v)
    pltpu.sync_copy(dst_h.at[s], dst_v)
    pltpu.sync_copy(w_h.at[s], w_v)
    pltpu.sync_copy(dis_h.at[s], dis_v)

    @pl.loop(0, SUB)
    def _z(r):
        for fc in range(FC):
            zbuf[r, pl.ds(fc * LANES, LANES)] = zero16

    for q in range(NSUB):
        n0 = s * NPT + q * SUB
        pltpu.sync_copy(zbuf, acc_s.at[pl.ds(n0, SUB), :])
    plsc.subcore_barrier()

    def edge_phase(gsrc):
        for b in range(GAHEAD):
            pltpu.async_copy(gsrc.at[src_v.at[b]], bufs[b], gsems[b])

        @pl.loop(0, NCHUNK, step=RING)
        def _chunks(j):
            for b in range(RING):
                jj = j + b
                pltpu.make_async_copy(gsrc.at[src_v.at[jj]], bufs[b],
                                      gsems[b]).wait()
                pltpu.sync_copy(bufs[b].at[pl.ds(0, 1), :], acc_s.at[pl.ds(0, 1), :])
                nb = (b + GAHEAD) % RING

                @pl.when(jj + GAHEAD < NCHUNK)
                def _():
                    pltpu.async_copy(gsrc.at[src_v.at[jj + GAHEAD]], bufs[nb],
                                     gsems[nb])


    def node_pass(final):
        scale_v = dis_v if final else w_v
        resid_h = u_h if final else g0_h
        dest_h = out_h if final else g_h
        for q in range(NSUB):
            n0 = s * NPT + q * SUB
            pltpu.sync_copy(acc_s.at[pl.ds(n0, SUB), :], buf0)
            pltpu.sync_copy(resid_h.at[c].at[pl.ds(n0, SUB), :], buf1)

            @pl.loop(0, SUB)
            def _rows(r):
                wsp = plsc.load_gather(
                    scale_v, [jnp.full((LANES,), q * SUB, i32) + r])
                for fc in range(FC):
                    sl = pl.ds(fc * LANES, LANES)
                    buf0[r, sl] = 0.9 * wsp * buf0[r, sl] + 0.1 * buf1[r, sl]

            pltpu.sync_copy(buf0, dest_h.at[c].at[pl.ds(n0, SUB), :])
            if not final:
                pltpu.sync_copy(zbuf, acc_s.at[pl.ds(n0, SUB), :])

    for it in range(DEPTH):
        edge_phase(g0_h.at[c] if it == 0 else g_h.at[c])
        plsc.subcore_barrier()
        node_pass(final=(it == DEPTH - 1))
        plsc.subcore_barrier()


def _diff_call(W, g0s, us, w16, dis16, src3, dst3):
    RING = 4 if W == 64 else 8
    GAHEAD = RING // 2
    k = pl.kernel(
        functools.partial(_diff_body, W, RING, GAHEAD),
        out_type=[jax.ShapeDtypeStruct((NC, NP, W), f32),
                  jax.ShapeDtypeStruct((NC, NP, W), f32)],
        mesh=_mesh(),
        compiler_params=pltpu.CompilerParams(needs_layout_passes=False,
                                             use_tc_tiling_on_sc=False),
        scratch_types=[
            pltpu.VMEM_SHARED((NP, W), f32),
            pltpu.VMEM((NCHUNK, ECHUNK), i32),
            pltpu.VMEM((NCHUNK, ECHUNK), i32),
            tuple(pltpu.VMEM((SUB, W), f32) for _ in range(RING)),
            pltpu.VMEM((SUB, W), f32),
            pltpu.VMEM((NPT,), f32),
            pltpu.VMEM((NPT,), f32),
            tuple(pltpu.SemaphoreType.DMA for _ in range(RING)),
            tuple(pltpu.SemaphoreType.DMA for _ in range(RING)),
        ],
    )
    out, _ = k(g0s, us, w16, dis16, src3, dst3)
    return out


def _mlp_body(h_ref, x_ref, dis_ref, embT_ref, a1wT_ref, a1b_ref,
              a2w_ref, a2b_ref, w1a_ref, w1b_ref, b1_ref, w2_ref, b2_ref,
              z0_ref, gz_ref):
    h = h_ref[:]
    x = x_ref[:]
    a1wT = a1wT_ref[:]
    a2w = a2w_ref[:]
    cT = (jnp.dot(a1wT[:, 2:], embT_ref[:],
                  preferred_element_type=f32,
                  precision=lax.Precision.HIGHEST) + a1b_ref[:])
    a = jnp.zeros_like(h)
    for j in range(12):
        t = jnp.maximum(h * a1wT[j, 0] + x * a1wT[j, 1] + cT[j:j + 1, :], 0.0)
        a = a + t * a2w[j, 0]
    a = (a + a2b_ref[0, 0]) * 0.5
    z = jnp.maximum(jnp.dot(a, w1a_ref[:], preferred_element_type=f32,
                  precision=lax.Precision.HIGHEST)
                    + jnp.dot(x, w1b_ref[:], preferred_element_type=f32,
                  precision=lax.Precision.HIGHEST)
                    + b1_ref[:], 0.0)
    z = jnp.dot(z, w2_ref[:], preferred_element_type=f32,
                  precision=lax.Precision.HIGHEST) + b2_ref[:]
    z0_ref[:] = z
    gz_ref[:] = z * dis_ref[:]


def _mlp_call(h, xp, dis, embT, a1wT, a1b, a2w, a2b, w1a, w1b, b1, w2, b2):
    bn = 256
    grid = NP // bn
    full = lambda shape: pl.BlockSpec(shape, lambda i: tuple(0 for _ in shape))
    return pl.pallas_call(
        _mlp_body,
        grid=(grid,),
        in_specs=[
            pl.BlockSpec((bn, FEATS), lambda i: (i, 0)),
            pl.BlockSpec((bn, FEATS), lambda i: (i, 0)),
            pl.BlockSpec((bn, 1), lambda i: (i, 0)),
            full((8, FEATS)),
            full((12, 10)),
            full((12, 1)),
            full((12, 1)),
            full((1, 1)),
            full((FEATS, 64)),
            full((FEATS, 64)),
            full((1, 64)),
            full((64, 16)),
            full((1, 16)),
        ],
        out_specs=[
            pl.BlockSpec((bn, 16), lambda i: (i, 0)),
            pl.BlockSpec((bn, 16), lambda i: (i, 0)),
        ],
        out_shape=[
            jax.ShapeDtypeStruct((NP, 16), f32),
            jax.ShapeDtypeStruct((NP, 16), f32),
        ],
    )(h, xp, dis, embT, a1wT, a1b, a2w, a2b, w1a, w1b, b1, w2, b2)


def kernel(x, edges, emb, A1w, A1b, A2w, A2b, W1, b1, W2, b2):
    src = edges[0]
    dst = edges[1]
    pad = ((0, 0), (0, EPTP - EPT))
    src3 = jnp.pad(src.reshape(NS, EPT), pad, constant_values=PADV).reshape(
        NS, NCHUNK, ECHUNK)
    dst3 = jnp.pad(dst.reshape(NS, EPT), pad, constant_values=PADV).reshape(
        NS, NCHUNK, ECHUNK)

    degp = _deg_call(dst)
    d0 = degp[0].reshape(NP, 1)
    d1 = degp[1].reshape(NP, 1)
    xp = jnp.pad(x, ((0, NP - N), (0, 0)))
    dis, wv, g0 = _prep_call(d0, d1, xp)

    g0s = g0.reshape(NP, 2, 64).transpose(1, 0, 2)
    us = xp.reshape(NP, 2, 64).transpose(1, 0, 2)
    w16 = wv.reshape(NS, NPT)
    dis16 = dis.reshape(NS, NPT)

    h = _diff_call(64, g0s, us, w16, dis16, src3, dst3)
    hfull = jnp.concatenate([h[0], h[1]], axis=1)

    z0, gz = _mlp_call(
        hfull, xp, dis,
        emb.T, A1w.T, A1b.reshape(12, 1), A2w, A2b.reshape(1, 1),
        W1[:FEATS], W1[FEATS:], b1.reshape(1, 64), W2, b2.reshape(1, 16))

    gzs = jnp.broadcast_to(gz[None], (NC, NP, 16))
    zs = jnp.broadcast_to(z0[None], (NC, NP, 16))
    zout = _diff_call(16, gzs, zs, w16, dis16, src3, dst3)
    return zout[0, :N, :]


# speedup vs baseline: 1.3907x; 1.3907x over previous
import functools

import jax
import jax.numpy as jnp
from jax import lax
from jax.experimental import pallas as pl
from jax.experimental.pallas import tpu as pltpu
from jax.experimental.pallas import tpu_sc as plsc

N = 10000
NP = 10240
FEATS = 128
E = 320000
NC, NS, LANES = 2, 16, 16
NPT = NP // NS
SUB = 128
NSUB = NPT // SUB
ECHUNK = 128
EPT = E // NS
EPTP = 20480
NCHUNK = EPTP // ECHUNK
PADV = 10016
EPW = E // (NC * NS)
DEPTH = 10

_mesh = functools.partial(
    plsc.VectorSubcoreMesh, core_axis_name="c", subcore_axis_name="s",
    num_cores=NC, num_subcores=NS)

f32 = jnp.float32
i32 = jnp.int32


def _deg_body(dst_h, out_h, idx_v, hist_v, cbuf, dbuf, hall_s):
    c = lax.axis_index("c")
    s = lax.axis_index("s")
    w = s * NC + c
    zero16 = jnp.zeros((LANES,), f32)
    one16 = jnp.ones((LANES,), f32)

    @pl.loop(0, NP // LANES)
    def _zero(r):
        hist_v[pl.ds(r * LANES, LANES)] = zero16

    pltpu.sync_copy(dst_h.at[pl.ds(w * EPW, EPW)], idx_v)

    @pl.loop(0, EPW // LANES)
    def _scat(i):
        idx16 = idx_v[pl.ds(i * LANES, LANES)]
        plsc.addupdate_scatter(hist_v, [idx16], one16)

    pltpu.sync_copy(hist_v, hall_s.at[s])
    plsc.subcore_barrier()

    @pl.loop(0, NPT // LANES)
    def _zd(r):
        dbuf[pl.ds(r * LANES, LANES)] = zero16

    for t in range(NS):
        pltpu.sync_copy(hall_s.at[t].at[pl.ds(s * NPT, NPT)], cbuf)

        @pl.loop(0, NPT // LANES)
        def _acc(r):
            sl = pl.ds(r * LANES, LANES)
            dbuf[sl] = dbuf[sl] + cbuf[sl]

    pltpu.sync_copy(dbuf, out_h.at[c].at[s])


def _deg_call(dst):
    k = pl.kernel(
        _deg_body,
        out_type=jax.ShapeDtypeStruct((NC, NS, NPT), f32),
        mesh=_mesh(),
        compiler_params=pltpu.CompilerParams(needs_layout_passes=False),
        scratch_types=[
            pltpu.VMEM((EPW,), i32),
            pltpu.VMEM((NP,), f32),
            pltpu.VMEM((NPT,), f32),
            pltpu.VMEM((NPT,), f32),
            pltpu.VMEM_SHARED((NS, NP), f32),
        ],
    )
    return k(dst)


def _prep_body(d0_ref, d1_ref, x_ref, dis_ref, w_ref, g0_ref):
    deg = d0_ref[:] + d1_ref[:]
    dis = jnp.where(deg > 0, lax.rsqrt(jnp.maximum(deg, 1e-12)), 0.0)
    dis_ref[:] = dis
    w_ref[:] = dis * dis
    g0_ref[:] = x_ref[:] * dis


def _prep_call(d0, d1, xp):
    bn = 256
    grid = NP // bn
    return pl.pallas_call(
        _prep_body,
        grid=(grid,),
        in_specs=[
            pl.BlockSpec((bn, 1), lambda i: (i, 0)),
            pl.BlockSpec((bn, 1), lambda i: (i, 0)),
            pl.BlockSpec((bn, FEATS), lambda i: (i, 0)),
        ],
        out_specs=[
            pl.BlockSpec((bn, 1), lambda i: (i, 0)),
            pl.BlockSpec((bn, 1), lambda i: (i, 0)),
            pl.BlockSpec((bn, FEATS), lambda i: (i, 0)),
        ],
        out_shape=[
            jax.ShapeDtypeStruct((NP, 1), f32),
            jax.ShapeDtypeStruct((NP, 1), f32),
            jax.ShapeDtypeStruct((NP, FEATS), f32),
        ],
    )(d0, d1, xp)


def _diff_body(W, RING, GA, R2, g0_h, u_h, w_h, dis_h, edg_h, out_h, g_h,
               acc_s, bufs, idxs, w_v, dis_v, isems, gsems, ssems):
    c = lax.axis_index("c")
    s = lax.axis_index("s")
    assert W % LANES == 0
    FC = W // LANES
    zero16 = jnp.zeros((LANES,), f32)
    buf0, buf1, zbuf = bufs[0], bufs[1], bufs[2]

    pltpu.sync_copy(w_h.at[s], w_v)
    pltpu.sync_copy(dis_h.at[s], dis_v)

    eh = edg_h.at[s]

    def zero_zbuf():
        @pl.loop(0, SUB)
        def _z(r):
            for fc in range(FC):
                zbuf[r, pl.ds(fc * LANES, LANES)] = zero16

    zero_zbuf()
    for q in range(NSUB):
        n0 = s * NPT + q * SUB
        pltpu.sync_copy(zbuf, acc_s.at[pl.ds(n0, SUB), :])
    plsc.subcore_barrier()

    def edge_phase(gsrc):
        for k in range(R2):
            pltpu.async_copy(eh.at[k], idxs[k], isems[k])
        for k in range(GA):
            pltpu.make_async_copy(eh.at[k], idxs[k], isems[k]).wait()
            pltpu.async_copy(gsrc.at[idxs[k].at[0]], bufs[k], gsems[k])

        @pl.loop(0, NCHUNK, step=RING)
        def _chunks(j):
            for b in range(RING):
                jj = j + b
                pltpu.make_async_copy(gsrc.at[idxs[b].at[0]], bufs[b],
                                      gsems[b]).wait()
                pltpu.async_copy(bufs[b], acc_s.at[idxs[b].at[1]], ssems[b],
                                 add=True)
                gb = (b + GA) % RING
                nb = (b + R2) % RING

                @pl.when(jj + GA < NCHUNK)
                def _():
                    pltpu.make_async_copy(eh.at[jj + GA], idxs[gb],
                                          isems[gb]).wait()
                    pltpu.async_copy(gsrc.at[idxs[gb].at[0]], bufs[gb],
                                     gsems[gb])

                @pl.when(jj + R2 < NCHUNK)
                def _():
                    @pl.when(jj + R2 >= RING)
                    def _():
                        pltpu.make_async_copy(
                            bufs[nb], acc_s.at[idxs[nb].at[1]],
                            ssems[nb]).wait()

                    pltpu.async_copy(eh.at[jj + R2], idxs[nb], isems[nb])

        for b in range(RING):
            pltpu.make_async_copy(bufs[b], acc_s.at[idxs[b].at[1]],
                                  ssems[b]).wait()

    def node_pass(final):
        zero_zbuf()
        scale_v = dis_v if final else w_v
        resid_h = u_h if final else g0_h
        dest_h = out_h if final else g_h
        for q in range(NSUB):
            n0 = s * NPT + q * SUB
            pltpu.sync_copy(acc_s.at[pl.ds(n0, SUB), :], buf0)
            pltpu.sync_copy(resid_h.at[c].at[pl.ds(n0, SUB), :], buf1)

            @pl.loop(0, SUB)
            def _rows(r):
                wsp = plsc.load_gather(
                    scale_v, [jnp.full((LANES,), q * SUB, i32) + r])
                for fc in range(FC):
                    sl = pl.ds(fc * LANES, LANES)
                    buf0[r, sl] = 0.9 * wsp * buf0[r, sl] + 0.1 * buf1[r, sl]

            pltpu.sync_copy(buf0, dest_h.at[c].at[pl.ds(n0, SUB), :])
            if not final:
                pltpu.sync_copy(zbuf, acc_s.at[pl.ds(n0, SUB), :])

    for it in range(DEPTH):
        edge_phase(g0_h.at[c] if it == 0 else g_h.at[c])
        plsc.subcore_barrier()
        node_pass(final=(it == DEPTH - 1))
        plsc.subcore_barrier()


def _diff_call(W, g0s, us, w16, dis16, edg):
    RING, GA, R2 = 10, 4, 7
    k = pl.kernel(
        functools.partial(_diff_body, W, RING, GA, R2),
        out_type=[jax.ShapeDtypeStruct((NC, NP, W), f32),
                  jax.ShapeDtypeStruct((NC, NP, W), f32)],
        mesh=_mesh(),
        compiler_params=pltpu.CompilerParams(needs_layout_passes=False,
                                             use_tc_tiling_on_sc=False),
        scratch_types=[
            pltpu.VMEM_SHARED((NP, W), f32),
            tuple(pltpu.VMEM((SUB, W), f32) for _ in range(RING)),
            tuple(pltpu.VMEM((2, ECHUNK), i32) for _ in range(RING)),
            pltpu.VMEM((NPT,), f32),
            pltpu.VMEM((NPT,), f32),
            tuple(pltpu.SemaphoreType.DMA for _ in range(RING)),
            tuple(pltpu.SemaphoreType.DMA for _ in range(RING)),
            tuple(pltpu.SemaphoreType.DMA for _ in range(RING)),
        ],
    )
    out, _ = k(g0s, us, w16, dis16, edg)
    return out


def _mlp_body(h_ref, x_ref, dis_ref, embT_ref, a1wT_ref, a1b_ref,
              a2w_ref, a2b_ref, w1a_ref, w1b_ref, b1_ref, w2_ref, b2_ref,
              z0_ref, gz_ref):
    h = h_ref[:]
    x = x_ref[:]
    a1wT = a1wT_ref[:]
    a2w = a2w_ref[:]
    cT = (jnp.dot(a1wT[:, 2:], embT_ref[:],
                  preferred_element_type=f32,
                  precision=lax.Precision.HIGHEST) + a1b_ref[:])
    a = jnp.zeros_like(h)
    for j in range(12):
        t = jnp.maximum(h * a1wT[j, 0] + x * a1wT[j, 1] + cT[j:j + 1, :], 0.0)
        a = a + t * a2w[j, 0]
    a = (a + a2b_ref[0, 0]) * 0.5
    z = jnp.maximum(jnp.dot(a, w1a_ref[:], preferred_element_type=f32,
                  precision=lax.Precision.HIGHEST)
                    + jnp.dot(x, w1b_ref[:], preferred_element_type=f32,
                  precision=lax.Precision.HIGHEST)
                    + b1_ref[:], 0.0)
    z = jnp.dot(z, w2_ref[:], preferred_element_type=f32,
                  precision=lax.Precision.HIGHEST) + b2_ref[:]
    z0_ref[:] = z
    gz_ref[:] = z * dis_ref[:]


def _mlp_call(h, xp, dis, embT, a1wT, a1b, a2w, a2b, w1a, w1b, b1, w2, b2):
    bn = 256
    grid = NP // bn
    full = lambda shape: pl.BlockSpec(shape, lambda i: tuple(0 for _ in shape))
    return pl.pallas_call(
        _mlp_body,
        grid=(grid,),
        in_specs=[
            pl.BlockSpec((bn, FEATS), lambda i: (i, 0)),
            pl.BlockSpec((bn, FEATS), lambda i: (i, 0)),
            pl.BlockSpec((bn, 1), lambda i: (i, 0)),
            full((8, FEATS)),
            full((12, 10)),
            full((12, 1)),
            full((12, 1)),
            full((1, 1)),
            full((FEATS, 64)),
            full((FEATS, 64)),
            full((1, 64)),
            full((64, 16)),
            full((1, 16)),
        ],
        out_specs=[
            pl.BlockSpec((bn, 16), lambda i: (i, 0)),
            pl.BlockSpec((bn, 16), lambda i: (i, 0)),
        ],
        out_shape=[
            jax.ShapeDtypeStruct((NP, 16), f32),
            jax.ShapeDtypeStruct((NP, 16), f32),
        ],
    )(h, xp, dis, embT, a1wT, a1b, a2w, a2b, w1a, w1b, b1, w2, b2)


def kernel(x, edges, emb, A1w, A1b, A2w, A2b, W1, b1, W2, b2):
    src = edges[0]
    dst = edges[1]
    pad = ((0, 0), (0, EPTP - EPT))
    src3 = jnp.pad(src.reshape(NS, EPT), pad, constant_values=PADV).reshape(
        NS, NCHUNK, ECHUNK)
    dst3 = jnp.pad(dst.reshape(NS, EPT), pad, constant_values=PADV).reshape(
        NS, NCHUNK, ECHUNK)
    edg = jnp.stack([src3, dst3], axis=2)

    degp = _deg_call(dst)
    d0 = degp[0].reshape(NP, 1)
    d1 = degp[1].reshape(NP, 1)
    xp = jnp.pad(x, ((0, NP - N), (0, 0)))
    dis, wv, g0 = _prep_call(d0, d1, xp)

    g0s = g0.reshape(NP, 2, 64).transpose(1, 0, 2)
    us = xp.reshape(NP, 2, 64).transpose(1, 0, 2)
    w16 = wv.reshape(NS, NPT)
    dis16 = dis.reshape(NS, NPT)

    h = _diff_call(64, g0s, us, w16, dis16, edg)
    hfull = jnp.concatenate([h[0], h[1]], axis=1)

    z0, gz = _mlp_call(
        hfull, xp, dis,
        emb.T, A1w.T, A1b.reshape(12, 1), A2w, A2b.reshape(1, 1),
        W1[:FEATS], W1[FEATS:], b1.reshape(1, 64), W2, b2.reshape(1, 16))

    gzs = jnp.broadcast_to(gz[None], (NC, NP, 16))
    zs = jnp.broadcast_to(z0[None], (NC, NP, 16))
    zout = _diff_call(16, gzs, zs, w16, dis16, edg)
    return zout[0, :N, :]
